# Initial kernel scaffold; baseline (speedup 1.0000x reference)
import jax
import jax.numpy as jnp
from jax.experimental import pallas as pl


def kernel(d, mask_d, embed, p_vector):
    raise NotImplementedError("write your pallas kernel here")


# speedup vs baseline: 1.1743x; 1.1743x over previous
import functools

import jax
import jax.numpy as jnp
from jax import lax
from jax.experimental import pallas as pl
from jax.experimental.pallas import tpu as pltpu
from jax.experimental.pallas import tpu_sc as plsc


def _embed_dot_p(embed, p_vector):
    V, E = embed.shape
    ROWS = 5000
    assert V % ROWS == 0
    NB = V // ROWS
    p8 = jnp.broadcast_to(p_vector.astype(jnp.float32)[None, :], (8, E))

    def body(e_ref, p_ref, o_ref):
        c = lax.dot_general(
            p_ref[...], e_ref[...],
            (((1,), (1,)), ((), ())),
            preferred_element_type=jnp.float32,
        )
        o_ref[0, 0, :] = c[0, :]

    s3 = pl.pallas_call(
        body,
        grid=(NB,),
        in_specs=[
            pl.BlockSpec((ROWS, E), lambda b: (b, 0)),
            pl.BlockSpec((8, E), lambda b: (0, 0)),
        ],
        out_specs=pl.BlockSpec((1, 1, ROWS), lambda b: (b, 0, 0)),
        out_shape=jax.ShapeDtypeStruct((NB, 1, ROWS), jnp.float32),
    )(embed, p8)
    return s3.reshape(V)


def _sc_pool(d, mask_d, s):
    B, L = d.shape
    info = plsc.get_sparse_core_info()
    NC, NS = info.num_cores, info.num_subcores
    NW = NC * NS
    RPW = B // NW
    G = RPW // 16
    assert B % NW == 0 and RPW % 16 == 0

    d_t = d.astype(jnp.int32).reshape(NW, RPW, L).swapaxes(1, 2).reshape(NW * L, RPW)
    m_t = mask_d.astype(jnp.float32).reshape(NW, RPW, L).swapaxes(1, 2).reshape(NW * L, RPW)

    mesh = plsc.VectorSubcoreMesh(core_axis_name="c", subcore_axis_name="s")

    @functools.partial(
        pl.kernel,
        mesh=mesh,
        out_type=jax.ShapeDtypeStruct((B,), jnp.float32),
        scratch_types=[
            pltpu.VMEM((L, RPW), jnp.int32),
            pltpu.VMEM((L, RPW), jnp.float32),
            pltpu.VMEM((L, RPW), jnp.float32),
            pltpu.VMEM((RPW,), jnp.float32),
            pltpu.SemaphoreType.DMA,
        ],
    )
    def k(d_hbm, m_hbm, s_hbm, out_hbm, idx_v, val_v, msk_v, out_v, sem):
        w = lax.axis_index("s") * NC + lax.axis_index("c")
        base = w * L
        pltpu.sync_copy(d_hbm.at[pl.ds(base, L)], idx_v)
        pltpu.sync_copy(m_hbm.at[pl.ds(base, L)], msk_v)

        CH = 8
        def gather_chunk(c, carry):
            t0 = c * CH
            copies = [
                pltpu.async_copy(
                    s_hbm.at[idx_v.at[t0 + j]], val_v.at[t0 + j], sem
                )
                for j in range(CH)
            ]
            for cp in copies:
                cp.wait()
            return carry

        lax.fori_loop(0, L // CH, gather_chunk, 0)

        zero = jnp.zeros((16,), jnp.float32)

        def body(t, accs):
            new = []
            for g in range(G):
                sv = val_v[t, pl.ds(g * 16, 16)]
                mv = msk_v[t, pl.ds(g * 16, 16)]
                new.append(accs[2 * g] + sv * mv)
                new.append(accs[2 * g + 1] + mv)
            return tuple(new)

        accs = lax.fori_loop(0, L, body, tuple(zero for _ in range(2 * G)))
        for g in range(G):
            r = accs[2 * g] / accs[2 * g + 1]
            out_v[pl.ds(g * 16, 16)] = 1.0 / (1.0 + jnp.exp(-r))
        pltpu.sync_copy(out_v, out_hbm.at[pl.ds(w * RPW, RPW)])

    return k(d_t, m_t, s)


def kernel(d, mask_d, embed, p_vector):
    s = _embed_dot_p(embed, p_vector)
    return _sc_pool(d, mask_d, s)


# speedup vs baseline: 1.2513x; 1.0655x over previous
import functools

import jax
import jax.numpy as jnp
from jax import lax
from jax.experimental import pallas as pl
from jax.experimental.pallas import tpu as pltpu
from jax.experimental.pallas import tpu_sc as plsc


def _embed_dot_p(embed, p_vector):
    V, E = embed.shape
    ROWS = 10000
    assert V % ROWS == 0
    NB = V // ROWS
    p8 = jnp.broadcast_to(p_vector.astype(jnp.float32)[None, :], (8, E))

    def body(e_ref, p_ref, o_ref):
        c = lax.dot_general(
            p_ref[...], e_ref[...],
            (((1,), (1,)), ((), ())),
            preferred_element_type=jnp.float32,
        )
        o_ref[0, 0, :] = c[0, :]

    s3 = pl.pallas_call(
        body,
        grid=(NB,),
        in_specs=[
            pl.BlockSpec((ROWS, E), lambda b: (b, 0)),
            pl.BlockSpec((8, E), lambda b: (0, 0)),
        ],
        out_specs=pl.BlockSpec((1, 1, ROWS), lambda b: (b, 0, 0)),
        out_shape=jax.ShapeDtypeStruct((NB, 1, ROWS), jnp.float32),
    )(embed, p8)
    return s3.reshape(V)


def _sc_pool(d, mask_d, s):
    B, L = d.shape
    info = plsc.get_sparse_core_info()
    NC, NS = info.num_cores, info.num_subcores
    NW = NC * NS
    RPW = B // NW
    G = RPW // 16
    assert B % NW == 0 and RPW % 16 == 0

    d_t = d.astype(jnp.int32).reshape(NW, RPW, L).swapaxes(1, 2).reshape(NW * L, RPW)
    m_t = mask_d.astype(jnp.float32).reshape(NW, RPW, L).swapaxes(1, 2).reshape(NW * L, RPW)

    mesh = plsc.VectorSubcoreMesh(core_axis_name="c", subcore_axis_name="s")

    @functools.partial(
        pl.kernel,
        mesh=mesh,
        out_type=jax.ShapeDtypeStruct((B,), jnp.float32),
        scratch_types=[
            pltpu.VMEM((L, RPW), jnp.int32),
            pltpu.VMEM((L, RPW), jnp.float32),
            pltpu.VMEM((L, RPW), jnp.float32),
            pltpu.VMEM((RPW,), jnp.float32),
            pltpu.SemaphoreType.DMA,
        ],
    )
    def k(d_hbm, m_hbm, s_hbm, out_hbm, idx_v, val_v, msk_v, out_v, sem):
        w = lax.axis_index("s") * NC + lax.axis_index("c")
        base = w * L
        pltpu.sync_copy(d_hbm.at[pl.ds(base, L)], idx_v)
        pltpu.sync_copy(m_hbm.at[pl.ds(base, L)], msk_v)

        CH = 8
        def gather_chunk(c, carry):
            t0 = c * CH
            copies = [
                pltpu.async_copy(
                    s_hbm.at[idx_v.at[t0 + j]], val_v.at[t0 + j], sem
                )
                for j in range(CH)
            ]
            for cp in copies:
                cp.wait()
            return carry

        lax.fori_loop(0, L // CH, gather_chunk, 0)

        zero = jnp.zeros((16,), jnp.float32)

        def body(t, accs):
            new = []
            for g in range(G):
                sv = val_v[t, pl.ds(g * 16, 16)]
                mv = msk_v[t, pl.ds(g * 16, 16)]
                new.append(accs[2 * g] + sv * mv)
                new.append(accs[2 * g + 1] + mv)
            return tuple(new)

        accs = lax.fori_loop(0, L, body, tuple(zero for _ in range(2 * G)))
        for g in range(G):
            r = accs[2 * g] / accs[2 * g + 1]
            out_v[pl.ds(g * 16, 16)] = 1.0 / (1.0 + jnp.exp(-r))
        pltpu.sync_copy(out_v, out_hbm.at[pl.ds(w * RPW, RPW)])

    return k(d_t, m_t, s)


def kernel(d, mask_d, embed, p_vector):
    s = _embed_dot_p(embed, p_vector)
    return _sc_pool(d, mask_d, s)
